# baseline (speedup 1.0000x reference)
import functools

import jax
import jax.numpy as jnp
from jax import lax
from jax.experimental import pallas as pl
from jax.experimental.pallas import tpu as pltpu
from jax.experimental.pallas import tpu_sc as plsc

N = 10000
D = 128
CLS = 64
E = 320000
NCORE = 2
NSUB = 16
NW = NCORE * NSUB
NPAD = 10240
RPT = NPAD // NSUB
CH = 80
EPW = E // NW
NCH = EPW // CH
BPW = NPAD // NW
BCH = BPW // CH

_mesh = lambda: plsc.VectorSubcoreMesh(
    core_axis_name="c", subcore_axis_name="s",
    num_cores=NCORE, num_subcores=NSUB)


@functools.lru_cache(maxsize=None)
def _make_embdeg():
  @functools.partial(
      pl.kernel,
      out_type=(jax.ShapeDtypeStruct((NPAD, D), jnp.float32),
                jax.ShapeDtypeStruct((2, NPAD, 16), jnp.float32)),
      mesh=_mesh(),
      scratch_types=[
          pltpu.VMEM((BCH, CH), jnp.int32),
          pltpu.VMEM((BCH, CH, D), jnp.float32),
          pltpu.VMEM((8, CH), jnp.int32),
          pltpu.VMEM((8, CH, 16), jnp.float32),
          pltpu.VMEM_SHARED((NPAD, 16), jnp.float32),
          pltpu.SemaphoreType.DMA,
          pltpu.SemaphoreType.DMA,
          pltpu.SemaphoreType.DMA,
          pltpu.SemaphoreType.DMA,
      ],
      compiler_params=pltpu.CompilerParams(use_tc_tiling_on_sc=False))
  def _embdeg(emb_h, idx_h, col_h, w_h, zdeg_h, h0_h, degp_h,
              idxv, rowsv, colv, wc, dacc, gsem, osem, csem, ssem):
    c = lax.axis_index("c")
    s = lax.axis_index("s")
    wid = s * NCORE + c

    @pl.when(s == 0)
    def _z():
        pltpu.sync_copy(zdeg_h, dacc)
    plsc.subcore_barrier()

    pltpu.sync_copy(idx_h.at[pl.ds(wid * BCH, BCH)], idxv)
    for j in range(BCH):
        pltpu.async_copy(emb_h.at[idxv.at[j]], rowsv.at[j], gsem)

    def issue_cw(k, slot):
        pltpu.async_copy(col_h.at[wid * NCH + k], colv.at[slot], csem)
        pltpu.async_copy(w_h.at[wid * NCH + k], wc.at[slot], csem)

    def wait_cw(k, slot):
        pltpu.make_async_copy(col_h.at[wid * NCH + k], colv.at[slot],
                              csem).wait()
        pltpu.make_async_copy(w_h.at[wid * NCH + k], wc.at[slot],
                              csem).wait()

    for k in range(5):
        issue_cw(k, k)

    for j in range(BCH):
        pltpu.make_async_copy(emb_h.at[idxv.at[j]], rowsv.at[j], gsem).wait()
        pltpu.async_copy(rowsv.at[j],
                         h0_h.at[pl.ds(wid * BPW + j * CH, CH)], osem)

    def dstep(k, _):
        i8 = lax.rem(k, 8)
        @pl.when(k > 2)
        def _ws():
            pltpu.make_async_copy(
                wc.at[lax.rem(k - 3, 8)],
                dacc.at[colv.at[lax.rem(k - 3, 8)]], ssem).wait()
        wait_cw(k, i8)
        @pl.when(k + 5 < NCH)
        def _i():
            issue_cw(k + 5, lax.rem(k + 5, 8))
        pltpu.async_copy(wc.at[i8], dacc.at[colv.at[i8]], ssem, add=True)
        return _
    lax.fori_loop(0, NCH, dstep, 0)
    for k in range(NCH - 3, NCH):
        pltpu.make_async_copy(wc.at[k % 8], dacc.at[colv.at[k % 8]],
                              ssem).wait()
    for j in range(BCH):
        pltpu.make_async_copy(rowsv.at[j],
                              h0_h.at[pl.ds(wid * BPW + j * CH, CH)],
                              osem).wait()
    plsc.subcore_barrier()

    pltpu.sync_copy(dacc.at[pl.ds(s * RPT, RPT)], degp_h.at[c, pl.ds(s * RPT, RPT)])

  return _embdeg


@functools.lru_cache(maxsize=None)
def _make_agg(d):
    nv = d // 16

    @functools.partial(
        pl.kernel,
        out_type=jax.ShapeDtypeStruct((2, NPAD, d), jnp.float32),
        mesh=_mesh(),
        scratch_types=[
            pltpu.VMEM((4, CH), jnp.int32),
            pltpu.VMEM((4, CH), jnp.int32),
            pltpu.VMEM((4, CH), jnp.float32),
            pltpu.VMEM((2, CH, d), jnp.float32),
            pltpu.VMEM((2, CH, d), jnp.float32),
            pltpu.VMEM_SHARED((NPAD, d), jnp.float32),
            pltpu.SemaphoreType.DMA,
            pltpu.SemaphoreType.DMA,
            pltpu.SemaphoreType.DMA,
        ],
        compiler_params=pltpu.CompilerParams(use_tc_tiling_on_sc=False))
    def agg(g_h, row_h, col_h, w_h, out_h, rowv, colv, wv, rbuf, sbuf, acc,
            isem, gsem, ssem):
        c = lax.axis_index("c")
        s = lax.axis_index("s")
        wid = s * NCORE + c
        base = wid * NCH

        def zrow(i, _):
            for j in range(nv):
                rbuf[0, i, pl.ds(j * 16, 16)] = jnp.zeros((16,), jnp.float32)
            return _
        lax.fori_loop(0, CH, zrow, 0)
        for t in range(RPT // CH):
            pltpu.sync_copy(rbuf.at[0], acc.at[pl.ds(s * RPT + t * CH, CH)])
        plsc.subcore_barrier()

        def issue_idx(k, slot):
            pltpu.async_copy(row_h.at[base + k], rowv.at[slot], isem)
            pltpu.async_copy(col_h.at[base + k], colv.at[slot], isem)
            pltpu.async_copy(w_h.at[base + k], wv.at[slot], isem)

        def wait_idx(k, slot):
            pltpu.make_async_copy(row_h.at[base + k], rowv.at[slot], isem).wait()
            pltpu.make_async_copy(col_h.at[base + k], colv.at[slot], isem).wait()
            pltpu.make_async_copy(w_h.at[base + k], wv.at[slot], isem).wait()

        issue_idx(0, 0)
        issue_idx(1, 1)
        wait_idx(0, 0)
        pltpu.async_copy(g_h.at[rowv.at[0]], rbuf.at[0], gsem)

        def step(k, _):
            b = lax.rem(k, 2)
            nb = 1 - b
            i4 = lax.rem(k, 4)
            @pl.when(k > 1)
            def _ws():
                pltpu.make_async_copy(
                    sbuf.at[b], acc.at[colv.at[lax.rem(k - 2, 4)]],
                    ssem).wait()
            @pl.when(k + 1 < NCH)
            def _g():
                i41 = lax.rem(k + 1, 4)
                wait_idx(k + 1, i41)
                pltpu.async_copy(g_h.at[rowv.at[i41]], rbuf.at[nb], gsem)
            @pl.when(k + 2 < NCH)
            def _i():
                issue_idx(k + 2, lax.rem(k + 2, 4))
            pltpu.make_async_copy(g_h.at[rowv.at[i4]], rbuf.at[b], gsem).wait()
            wk = wv.at[i4]
            rb = rbuf.at[b]
            sb = sbuf.at[b]
            for g in range(CH // 16):
                wgrp = wk[pl.ds(g * 16, 16)]
                for l in range(16):
                    ws = jnp.full((16,), wgrp[l], jnp.float32)
                    i = g * 16 + l
                    for j in range(nv):
                        sl = pl.ds(j * 16, 16)
                        sb[i, sl] = rb[i, sl] * ws
            pltpu.async_copy(sb, acc.at[colv.at[i4]], ssem, add=True)
            return _
        lax.fori_loop(0, NCH, step, 0)
        pltpu.make_async_copy(sbuf.at[(NCH - 2) % 2],
                              acc.at[colv.at[(NCH - 2) % 4]], ssem).wait()
        pltpu.make_async_copy(sbuf.at[(NCH - 1) % 2],
                              acc.at[colv.at[(NCH - 1) % 4]], ssem).wait()
        plsc.subcore_barrier()

        pltpu.sync_copy(acc.at[pl.ds(s * RPT, RPT)],
                        out_h.at[c, pl.ds(s * RPT, RPT)])

    return agg


_BLK = 512
_GRID = NPAD // _BLK
_PREC = lax.Precision.HIGHEST


def _tc1_body(h0_ref, w1_ref, degp_ref, g1_ref, dinv_ref):
    deg = degp_ref[0, :, 0:1] + degp_ref[1, :, 0:1] + 1.0
    dinv = lax.rsqrt(deg)
    hw1 = jnp.dot(h0_ref[...], w1_ref[...],
                  preferred_element_type=jnp.float32, precision=_PREC)
    g1_ref[...] = hw1 * dinv
    dinv_ref[...] = dinv


def _tc1(h0, W1, degp):
    return pl.pallas_call(
        _tc1_body,
        grid=(_GRID,),
        in_specs=[
            pl.BlockSpec((_BLK, D), lambda i: (i, 0)),
            pl.BlockSpec((D, D), lambda i: (0, 0)),
            pl.BlockSpec((2, _BLK, 16), lambda i: (0, i, 0)),
        ],
        out_specs=[
            pl.BlockSpec((_BLK, D), lambda i: (i, 0)),
            pl.BlockSpec((_BLK, 1), lambda i: (i, 0)),
        ],
        out_shape=[
            jax.ShapeDtypeStruct((NPAD, D), jnp.float32),
            jax.ShapeDtypeStruct((NPAD, 1), jnp.float32),
        ],
    )(h0, W1, degp)


def _tc2_body(aggp_ref, g1_ref, dinv_ref, b1_ref, w2_ref, g2_ref):
    dinv = dinv_ref[...]
    pre = (aggp_ref[0] + aggp_ref[1] + g1_ref[...]) * dinv + b1_ref[...]
    h1 = jnp.maximum(pre, 0.0)
    hw2 = jnp.dot(h1, w2_ref[...],
                  preferred_element_type=jnp.float32, precision=_PREC)
    g2_ref[...] = hw2 * dinv


def _tc2(aggp, g1, dinv, b1, W2):
    return pl.pallas_call(
        _tc2_body,
        grid=(_GRID,),
        in_specs=[
            pl.BlockSpec((2, _BLK, D), lambda i: (0, i, 0)),
            pl.BlockSpec((_BLK, D), lambda i: (i, 0)),
            pl.BlockSpec((_BLK, 1), lambda i: (i, 0)),
            pl.BlockSpec((1, D), lambda i: (0, 0)),
            pl.BlockSpec((D, CLS), lambda i: (0, 0)),
        ],
        out_specs=pl.BlockSpec((_BLK, CLS), lambda i: (i, 0)),
        out_shape=jax.ShapeDtypeStruct((NPAD, CLS), jnp.float32),
    )(aggp, g1, dinv, b1, W2)


def _tc3_body(aggp_ref, g2_ref, dinv_ref, b2_ref, out_ref):
    pre = (aggp_ref[0] + aggp_ref[1] + g2_ref[...]) * dinv_ref[...] + b2_ref[...]
    m = jnp.max(pre, axis=1, keepdims=True)
    ex = jnp.exp(pre - m)
    out_ref[...] = pre - m - jnp.log(jnp.sum(ex, axis=1, keepdims=True))


def _tc3(aggp, g2, dinv, b2):
    return pl.pallas_call(
        _tc3_body,
        grid=(_GRID,),
        in_specs=[
            pl.BlockSpec((2, _BLK, CLS), lambda i: (0, i, 0)),
            pl.BlockSpec((_BLK, CLS), lambda i: (i, 0)),
            pl.BlockSpec((_BLK, 1), lambda i: (i, 0)),
            pl.BlockSpec((1, CLS), lambda i: (0, 0)),
        ],
        out_specs=pl.BlockSpec((_BLK, CLS), lambda i: (i, 0)),
        out_shape=jax.ShapeDtypeStruct((NPAD, CLS), jnp.float32),
    )(aggp, g2, dinv, b2)


def kernel(x, edge_index, edge_attr, emb, W1, b1, W2, b2):
    row = edge_index[0]
    col = edge_index[1]
    xpad = jnp.concatenate([x[:, 0], jnp.zeros((NPAD - N,), x.dtype)])
    row2 = row.reshape(NW * NCH, CH)
    col2 = col.reshape(NW * NCH, CH)
    w2 = edge_attr.reshape(NW * NCH, CH)
    idx2 = xpad.reshape(NW * BCH, CH)
    w3 = jnp.broadcast_to(edge_attr.reshape(NW * NCH, CH, 1),
                          (NW * NCH, CH, 16))
    zdeg = jnp.zeros((NPAD, 16), jnp.float32)
    h0, degp = _make_embdeg()(emb, idx2, col2, w3, zdeg)
    g1, dinv = _tc1(h0, W1, degp)
    agg1 = _make_agg(D)(g1, row2, col2, w2)
    g2 = _tc2(agg1, g1, dinv, b1.reshape(1, D), W2)
    agg2 = _make_agg(CLS)(g2, row2, col2, w2)
    out = _tc3(agg2, g2, dinv, b2.reshape(1, CLS))
    return out[:N]


# speedup vs baseline: 1.5239x; 1.5239x over previous
import functools

import jax
import jax.numpy as jnp
from jax import lax
from jax.experimental import pallas as pl
from jax.experimental.pallas import tpu as pltpu
from jax.experimental.pallas import tpu_sc as plsc

N = 10000
D = 128
CLS = 64
E = 320000
NCORE = 2
NSUB = 16
NW = NCORE * NSUB
NPAD = 10240
RPT = NPAD // NSUB
CH = 80
EPW = E // NW
NCH = EPW // CH
BPW = NPAD // NW
BCH = BPW // CH

_mesh = lambda: plsc.VectorSubcoreMesh(
    core_axis_name="c", subcore_axis_name="s",
    num_cores=NCORE, num_subcores=NSUB)


@functools.lru_cache(maxsize=None)
def _make_embdeg():
  @functools.partial(
      pl.kernel,
      out_type=(jax.ShapeDtypeStruct((NPAD, D), jnp.float32),
                jax.ShapeDtypeStruct((2, NPAD, 16), jnp.float32)),
      mesh=_mesh(),
      scratch_types=[
          pltpu.VMEM((BCH, CH), jnp.int32),
          pltpu.VMEM((BCH, CH, D), jnp.float32),
          pltpu.VMEM((8, CH), jnp.int32),
          pltpu.VMEM((8, CH), jnp.float32),
          pltpu.VMEM((4, CH, 16), jnp.float32),
          pltpu.VMEM_SHARED((NPAD, 16), jnp.float32),
          pltpu.SemaphoreType.DMA,
          pltpu.SemaphoreType.DMA,
          pltpu.SemaphoreType.DMA,
          pltpu.SemaphoreType.DMA,
      ],
      compiler_params=pltpu.CompilerParams(use_tc_tiling_on_sc=False))
  def _embdeg(emb_h, idx_h, col_h, w_h, zdeg_h, h0_h, degp_h,
              idxv, rowsv, colv, wv, wr, dacc, gsem, osem, csem, ssem):
    c = lax.axis_index("c")
    s = lax.axis_index("s")
    wid = s * NCORE + c

    @pl.when(s == 0)
    def _z():
        pltpu.sync_copy(zdeg_h, dacc)
    plsc.subcore_barrier()

    pltpu.sync_copy(idx_h.at[pl.ds(wid * BCH, BCH)], idxv)
    for j in range(BCH):
        pltpu.async_copy(emb_h.at[idxv.at[j]], rowsv.at[j], gsem)

    def issue_cw(k, slot):
        pltpu.async_copy(col_h.at[wid * NCH + k], colv.at[slot], csem)
        pltpu.async_copy(w_h.at[wid * NCH + k], wv.at[slot], csem)

    def wait_cw(k, slot):
        pltpu.make_async_copy(col_h.at[wid * NCH + k], colv.at[slot],
                              csem).wait()
        pltpu.make_async_copy(w_h.at[wid * NCH + k], wv.at[slot],
                              csem).wait()

    for k in range(5):
        issue_cw(k, k)

    for j in range(BCH):
        pltpu.make_async_copy(emb_h.at[idxv.at[j]], rowsv.at[j], gsem).wait()
        pltpu.async_copy(rowsv.at[j],
                         h0_h.at[pl.ds(wid * BPW + j * CH, CH)], osem)

    def dstep(k, _):
        i8 = lax.rem(k, 8)
        i4 = lax.rem(k, 4)
        @pl.when(k > 2)
        def _ws():
            pltpu.make_async_copy(
                wr.at[lax.rem(k - 3, 4)],
                dacc.at[colv.at[lax.rem(k - 3, 8)]], ssem).wait()
        wait_cw(k, i8)
        @pl.when(k + 5 < NCH)
        def _i():
            issue_cw(k + 5, lax.rem(k + 5, 8))
        wk = wv.at[i8]
        wrk = wr.at[i4]
        for g in range(CH // 16):
            wgrp = wk[pl.ds(g * 16, 16)]
            for l in range(16):
                wrk[g * 16 + l, :] = jnp.full((16,), wgrp[l], jnp.float32)
        pltpu.async_copy(wrk, dacc.at[colv.at[i8]], ssem, add=True)
        return _
    lax.fori_loop(0, NCH, dstep, 0)
    for k in range(NCH - 3, NCH):
        pltpu.make_async_copy(wr.at[k % 4], dacc.at[colv.at[k % 8]],
                              ssem).wait()
    for j in range(BCH):
        pltpu.make_async_copy(rowsv.at[j],
                              h0_h.at[pl.ds(wid * BPW + j * CH, CH)],
                              osem).wait()
    plsc.subcore_barrier()

    pltpu.sync_copy(dacc.at[pl.ds(s * RPT, RPT)], degp_h.at[c, pl.ds(s * RPT, RPT)])

  return _embdeg


@functools.lru_cache(maxsize=None)
def _make_agg(d):
    nv = d // 16

    @functools.partial(
        pl.kernel,
        out_type=jax.ShapeDtypeStruct((2, NPAD, d), jnp.float32),
        mesh=_mesh(),
        scratch_types=[
            pltpu.VMEM((4, CH), jnp.int32),
            pltpu.VMEM((4, CH), jnp.int32),
            pltpu.VMEM((4, CH), jnp.float32),
            pltpu.VMEM((2, CH, d), jnp.float32),
            pltpu.VMEM((2, CH, d), jnp.float32),
            pltpu.VMEM_SHARED((NPAD, d), jnp.float32),
            pltpu.SemaphoreType.DMA,
            pltpu.SemaphoreType.DMA,
            pltpu.SemaphoreType.DMA,
        ],
        compiler_params=pltpu.CompilerParams(use_tc_tiling_on_sc=False))
    def agg(g_h, row_h, col_h, w_h, out_h, rowv, colv, wv, rbuf, sbuf, acc,
            isem, gsem, ssem):
        c = lax.axis_index("c")
        s = lax.axis_index("s")
        wid = s * NCORE + c
        base = wid * NCH

        def zrow(i, _):
            for j in range(nv):
                rbuf[0, i, pl.ds(j * 16, 16)] = jnp.zeros((16,), jnp.float32)
            return _
        lax.fori_loop(0, CH, zrow, 0)
        for t in range(RPT // CH):
            pltpu.sync_copy(rbuf.at[0], acc.at[pl.ds(s * RPT + t * CH, CH)])
        plsc.subcore_barrier()

        def issue_idx(k, slot):
            pltpu.async_copy(row_h.at[base + k], rowv.at[slot], isem)
            pltpu.async_copy(col_h.at[base + k], colv.at[slot], isem)
            pltpu.async_copy(w_h.at[base + k], wv.at[slot], isem)

        def wait_idx(k, slot):
            pltpu.make_async_copy(row_h.at[base + k], rowv.at[slot], isem).wait()
            pltpu.make_async_copy(col_h.at[base + k], colv.at[slot], isem).wait()
            pltpu.make_async_copy(w_h.at[base + k], wv.at[slot], isem).wait()

        issue_idx(0, 0)
        issue_idx(1, 1)
        wait_idx(0, 0)
        pltpu.async_copy(g_h.at[rowv.at[0]], rbuf.at[0], gsem)

        def step(k, _):
            b = lax.rem(k, 2)
            nb = 1 - b
            i4 = lax.rem(k, 4)
            @pl.when(k > 1)
            def _ws():
                pltpu.make_async_copy(
                    sbuf.at[b], acc.at[colv.at[lax.rem(k - 2, 4)]],
                    ssem).wait()
            @pl.when(k + 1 < NCH)
            def _g():
                i41 = lax.rem(k + 1, 4)
                wait_idx(k + 1, i41)
                pltpu.async_copy(g_h.at[rowv.at[i41]], rbuf.at[nb], gsem)
            @pl.when(k + 2 < NCH)
            def _i():
                issue_idx(k + 2, lax.rem(k + 2, 4))
            pltpu.make_async_copy(g_h.at[rowv.at[i4]], rbuf.at[b], gsem).wait()
            wk = wv.at[i4]
            rb = rbuf.at[b]
            sb = sbuf.at[b]
            for g in range(CH // 16):
                wgrp = wk[pl.ds(g * 16, 16)]
                for l in range(16):
                    ws = jnp.full((16,), wgrp[l], jnp.float32)
                    i = g * 16 + l
                    for j in range(nv):
                        sl = pl.ds(j * 16, 16)
                        sb[i, sl] = rb[i, sl] * ws
            pltpu.async_copy(sb, acc.at[colv.at[i4]], ssem, add=True)
            return _
        lax.fori_loop(0, NCH, step, 0)
        pltpu.make_async_copy(sbuf.at[(NCH - 2) % 2],
                              acc.at[colv.at[(NCH - 2) % 4]], ssem).wait()
        pltpu.make_async_copy(sbuf.at[(NCH - 1) % 2],
                              acc.at[colv.at[(NCH - 1) % 4]], ssem).wait()
        plsc.subcore_barrier()

        pltpu.sync_copy(acc.at[pl.ds(s * RPT, RPT)],
                        out_h.at[c, pl.ds(s * RPT, RPT)])

    return agg


_BLK = 512
_GRID = NPAD // _BLK
_PREC = lax.Precision.HIGHEST


def _tc1_body(h0_ref, w1_ref, degp_ref, g1_ref, dinv_ref):
    deg = degp_ref[0, :, 0:1] + degp_ref[1, :, 0:1] + 1.0
    dinv = lax.rsqrt(deg)
    hw1 = jnp.dot(h0_ref[...], w1_ref[...],
                  preferred_element_type=jnp.float32, precision=_PREC)
    g1_ref[...] = hw1 * dinv
    dinv_ref[...] = dinv


def _tc1(h0, W1, degp):
    return pl.pallas_call(
        _tc1_body,
        grid=(_GRID,),
        in_specs=[
            pl.BlockSpec((_BLK, D), lambda i: (i, 0)),
            pl.BlockSpec((D, D), lambda i: (0, 0)),
            pl.BlockSpec((2, _BLK, 16), lambda i: (0, i, 0)),
        ],
        out_specs=[
            pl.BlockSpec((_BLK, D), lambda i: (i, 0)),
            pl.BlockSpec((_BLK, 1), lambda i: (i, 0)),
        ],
        out_shape=[
            jax.ShapeDtypeStruct((NPAD, D), jnp.float32),
            jax.ShapeDtypeStruct((NPAD, 1), jnp.float32),
        ],
    )(h0, W1, degp)


def _tc2_body(aggp_ref, g1_ref, dinv_ref, b1_ref, w2_ref, g2_ref):
    dinv = dinv_ref[...]
    pre = (aggp_ref[0] + aggp_ref[1] + g1_ref[...]) * dinv + b1_ref[...]
    h1 = jnp.maximum(pre, 0.0)
    hw2 = jnp.dot(h1, w2_ref[...],
                  preferred_element_type=jnp.float32, precision=_PREC)
    g2_ref[...] = hw2 * dinv


def _tc2(aggp, g1, dinv, b1, W2):
    return pl.pallas_call(
        _tc2_body,
        grid=(_GRID,),
        in_specs=[
            pl.BlockSpec((2, _BLK, D), lambda i: (0, i, 0)),
            pl.BlockSpec((_BLK, D), lambda i: (i, 0)),
            pl.BlockSpec((_BLK, 1), lambda i: (i, 0)),
            pl.BlockSpec((1, D), lambda i: (0, 0)),
            pl.BlockSpec((D, CLS), lambda i: (0, 0)),
        ],
        out_specs=pl.BlockSpec((_BLK, CLS), lambda i: (i, 0)),
        out_shape=jax.ShapeDtypeStruct((NPAD, CLS), jnp.float32),
    )(aggp, g1, dinv, b1, W2)


def _tc3_body(aggp_ref, g2_ref, dinv_ref, b2_ref, out_ref):
    pre = (aggp_ref[0] + aggp_ref[1] + g2_ref[...]) * dinv_ref[...] + b2_ref[...]
    m = jnp.max(pre, axis=1, keepdims=True)
    ex = jnp.exp(pre - m)
    out_ref[...] = pre - m - jnp.log(jnp.sum(ex, axis=1, keepdims=True))


def _tc3(aggp, g2, dinv, b2):
    return pl.pallas_call(
        _tc3_body,
        grid=(_GRID,),
        in_specs=[
            pl.BlockSpec((2, _BLK, CLS), lambda i: (0, i, 0)),
            pl.BlockSpec((_BLK, CLS), lambda i: (i, 0)),
            pl.BlockSpec((_BLK, 1), lambda i: (i, 0)),
            pl.BlockSpec((1, CLS), lambda i: (0, 0)),
        ],
        out_specs=pl.BlockSpec((_BLK, CLS), lambda i: (i, 0)),
        out_shape=jax.ShapeDtypeStruct((NPAD, CLS), jnp.float32),
    )(aggp, g2, dinv, b2)


def kernel(x, edge_index, edge_attr, emb, W1, b1, W2, b2):
    row = edge_index[0]
    col = edge_index[1]
    xpad = jnp.concatenate([x[:, 0], jnp.zeros((NPAD - N,), x.dtype)])
    row2 = row.reshape(NW * NCH, CH)
    col2 = col.reshape(NW * NCH, CH)
    w2 = edge_attr.reshape(NW * NCH, CH)
    idx2 = xpad.reshape(NW * BCH, CH)
    zdeg = jnp.zeros((NPAD, 16), jnp.float32)
    h0, degp = _make_embdeg()(emb, idx2, col2, w2, zdeg)
    g1, dinv = _tc1(h0, W1, degp)
    agg1 = _make_agg(D)(g1, row2, col2, w2)
    g2 = _tc2(agg1, g1, dinv, b1.reshape(1, D), W2)
    agg2 = _make_agg(CLS)(g2, row2, col2, w2)
    out = _tc3(agg2, g2, dinv, b2.reshape(1, CLS))
    return out[:N]
